# Initial kernel scaffold; baseline (speedup 1.0000x reference)
import jax
import jax.numpy as jnp
from jax.experimental import pallas as pl


def kernel(gmap, atom, bonds, W_be, b_be, W_ae, b_ae, W_bu, b_bu, W_au, b_au, W_fc, b_fc):
    raise NotImplementedError("write your pallas kernel here")


# speedup vs baseline: 2.3250x; 2.3250x over previous
import functools

import jax
import jax.numpy as jnp
from jax import lax
from jax.experimental import pallas as pl
from jax.experimental.pallas import tpu as pltpu
from jax.experimental.pallas import tpu_sc as plsc

B = 10000
NNN = 32
E = B * NNN
NBF = 16
H = 128

NC = 2
NS = 16
NW = NC * NS
CHUNK = 128
NCHUNKS = E // CHUNK


def _gather_body(table_hbm, idx_hbm, out_hbm, idx_v, rows_v, sem):
    wid = lax.axis_index("s") * NC + lax.axis_index("c")
    n_mine = (NCHUNKS - 1 - wid) // NW + 1

    def step(k, carry):
        base = (wid + k * NW) * CHUNK
        pltpu.sync_copy(idx_hbm.at[pl.ds(base, CHUNK)], idx_v)
        pltpu.async_copy(table_hbm.at[idx_v], rows_v, sem).wait()
        pltpu.sync_copy(rows_v, out_hbm.at[pl.ds(base, CHUNK)])
        return carry

    lax.fori_loop(0, n_mine, step, 0)


_sc_gather = pl.kernel(
    _gather_body,
    out_type=jax.ShapeDtypeStruct((E, H), jnp.float32),
    mesh=plsc.VectorSubcoreMesh(
        core_axis_name="c", subcore_axis_name="s", num_cores=NC, num_subcores=NS
    ),
    scratch_types=[
        pltpu.VMEM((CHUNK,), jnp.int32),
        pltpu.VMEM((CHUNK, H), jnp.float32),
        pltpu.SemaphoreType.DMA,
    ],
)

BA = 400
EBLK = BA * NNN
GRID = B // BA

_dot = functools.partial(jnp.dot, preferred_element_type=jnp.float32)


def _proj_body(atom_ref, w1_ref, b1_ref, w2_ref, a1_ref, a2_ref):
    a = atom_ref[...]
    a1_ref[...] = _dot(a, w1_ref[...]) + b1_ref[...]
    a2_ref[...] = _dot(a, w2_ref[...])


def _edge_atom_stage(m, g, p1, ah, wau1, wau2, bau):
    t = jnp.tanh((m + g).reshape(BA, NNN, H) + p1[:, None, :])
    mean = jnp.sum(t, axis=1) * (1.0 / NNN)
    ah_new = jnp.maximum(_dot(mean, wau1) + _dot(ah, wau2) + bau, 0.0)
    return t, ah_new


def _layer0_body(bonds_ref, g_ref, a1_ref, atom_ref, w3_ref, wae1_ref,
                 wae2_ref, bae_ref, wb1_ref, bb_ref, wb2_ref,
                 bh_ref, ah_ref, p1_ref, p2_ref):
    m = _dot(bonds_ref[...], w3_ref[...])
    t, ah = _edge_atom_stage(m, g_ref[...], a1_ref[...], atom_ref[...],
                             wae1_ref[...], wae2_ref[...], bae_ref[...])
    bh_ref[...] = t.reshape(EBLK, H)
    ah_ref[...] = ah
    p1_ref[...] = _dot(ah, wb1_ref[...]) + bb_ref[...]
    p2_ref[...] = _dot(ah, wb2_ref[...])


def _conv_body(bhin_ref, g_ref, p1in_ref, ahin_ref, w3_ref, wau1_ref,
               wau2_ref, bau_ref, wb1_ref, bb_ref, wb2_ref,
               bh_ref, ah_ref, p1_ref, p2_ref):
    m = _dot(bhin_ref[...], w3_ref[...])
    t, ah = _edge_atom_stage(m, g_ref[...], p1in_ref[...], ahin_ref[...],
                             wau1_ref[...], wau2_ref[...], bau_ref[...])
    bh_ref[...] = t.reshape(EBLK, H)
    ah_ref[...] = ah
    p1_ref[...] = _dot(ah, wb1_ref[...]) + bb_ref[...]
    p2_ref[...] = _dot(ah, wb2_ref[...])


def _final_body(bhin_ref, g_ref, p1in_ref, ahin_ref, w3_ref, wau1_ref,
                wau2_ref, bau_ref, wfc_ref, bfc_ref, y_ref):
    m = _dot(bhin_ref[...], w3_ref[...])
    _, ah = _edge_atom_stage(m, g_ref[...], p1in_ref[...], ahin_ref[...],
                             wau1_ref[...], wau2_ref[...], bau_ref[...])
    z = _dot(ah, wfc_ref[...]) + bfc_ref[...]
    y_ref[...] = jnp.maximum(z, 0.0) + jnp.log1p(jnp.exp(-jnp.abs(z)))


def _edge_spec():
    return pl.BlockSpec((EBLK, H), lambda i: (i, 0))


def _atom_spec(width=H):
    return pl.BlockSpec((BA, width), lambda i: (i, 0))


def _full_spec(shape):
    return pl.BlockSpec(shape, lambda i: tuple(0 for _ in shape))


def _wspec():
    return pl.BlockSpec((H, H), lambda i: (0, 0))


def _bspec():
    return pl.BlockSpec((1, H), lambda i: (0, 0))


_params = pltpu.CompilerParams(dimension_semantics=("parallel",))


def _pc(body, in_specs, out_specs, out_shapes):
    return pl.pallas_call(
        body,
        grid=(GRID,),
        in_specs=in_specs,
        out_specs=out_specs,
        out_shape=out_shapes,
        compiler_params=_params,
    )


def kernel(gmap, atom, bonds, W_be, b_be, W_ae, b_ae, W_bu, b_bu, W_au, b_au,
           W_fc, b_fc):
    idx = gmap.astype(jnp.int32).reshape(E)
    bonds2 = bonds.reshape(E, NBF)

    wbe1, wbe2, wbe3 = W_be[:H], W_be[H:2 * H], W_be[2 * H:]
    wae1, wae2 = W_ae[:H], W_ae[H:]
    wbu1, wbu2, wbu3 = W_bu[:H], W_bu[H:2 * H], W_bu[2 * H:]
    wau1, wau2 = W_au[:H], W_au[H:]
    b_be2 = b_be.reshape(1, H)
    b_ae2 = b_ae.reshape(1, H)
    b_bu2 = b_bu.reshape(1, H)
    b_au2 = b_au.reshape(1, H)
    b_fc2 = b_fc.reshape(1, 1)

    a1, a2 = _pc(
        _proj_body,
        [_atom_spec(), _wspec(), _bspec(), _wspec()],
        [_atom_spec(), _atom_spec()],
        [jax.ShapeDtypeStruct((B, H), jnp.float32)] * 2,
    )(atom, wbe1, b_be2, wbe2)

    g = _sc_gather(a2, idx)

    edge_out = jax.ShapeDtypeStruct((E, H), jnp.float32)
    atom_out = jax.ShapeDtypeStruct((B, H), jnp.float32)

    bh, ah, p1, p2 = _pc(
        _layer0_body,
        [pl.BlockSpec((EBLK, NBF), lambda i: (i, 0)), _edge_spec(),
         _atom_spec(), _atom_spec(), pl.BlockSpec((NBF, H), lambda i: (0, 0)),
         _wspec(), _wspec(), _bspec(), _wspec(), _bspec(), _wspec()],
        [_edge_spec(), _atom_spec(), _atom_spec(), _atom_spec()],
        [edge_out, atom_out, atom_out, atom_out],
    )(bonds2, g, a1, atom, wbe3, wae1, wae2, b_ae2, wbu1, b_bu2, wbu2)

    for _ in range(2):
        g = _sc_gather(p2, idx)
        bh, ah, p1, p2 = _pc(
            _conv_body,
            [_edge_spec(), _edge_spec(), _atom_spec(), _atom_spec(),
             _wspec(), _wspec(), _wspec(), _bspec(), _wspec(), _bspec(),
             _wspec()],
            [_edge_spec(), _atom_spec(), _atom_spec(), _atom_spec()],
            [edge_out, atom_out, atom_out, atom_out],
        )(bh, g, p1, ah, wbu3, wau1, wau2, b_au2, wbu1, b_bu2, wbu2)

    g = _sc_gather(p2, idx)
    (y,) = _pc(
        _final_body,
        [_edge_spec(), _edge_spec(), _atom_spec(), _atom_spec(),
         _wspec(), _wspec(), _wspec(), _bspec(),
         pl.BlockSpec((H, 1), lambda i: (0, 0)),
         pl.BlockSpec((1, 1), lambda i: (0, 0))],
        [_atom_spec(1)],
        [jax.ShapeDtypeStruct((B, 1), jnp.float32)],
    )(bh, g, p1, ah, wbu3, wau1, wau2, b_au2, W_fc, b_fc2)

    return y


# speedup vs baseline: 2.6996x; 1.1612x over previous
import functools

import jax
import jax.numpy as jnp
from jax import lax
from jax.experimental import pallas as pl
from jax.experimental.pallas import tpu as pltpu
from jax.experimental.pallas import tpu_sc as plsc

B = 10000
NNN = 32
E = B * NNN
NBF = 16
H = 128

NSPLIT = 5
ASPLIT = B // NSPLIT
ESPLIT = ASPLIT * NNN

NC = 2
NS = 16
NW = NC * NS
CHUNK = 128
NCHUNKS = ESPLIT // CHUNK


def _gather_body(table_hbm, idx_hbm, out_hbm, idx_v, rows_v, sem):
    wid = lax.axis_index("s") * NC + lax.axis_index("c")
    n_mine = (NCHUNKS - 1 - wid) // NW + 1

    def step(k, carry):
        base = (wid + k * NW) * CHUNK
        pltpu.sync_copy(idx_hbm.at[pl.ds(base, CHUNK)], idx_v)
        pltpu.async_copy(table_hbm.at[idx_v], rows_v, sem).wait()
        pltpu.sync_copy(rows_v, out_hbm.at[pl.ds(base, CHUNK)])
        return carry

    lax.fori_loop(0, n_mine, step, 0)


_sc_gather = pl.kernel(
    _gather_body,
    out_type=jax.ShapeDtypeStruct((ESPLIT, H), jnp.float32),
    mesh=plsc.VectorSubcoreMesh(
        core_axis_name="c", subcore_axis_name="s", num_cores=NC, num_subcores=NS
    ),
    scratch_types=[
        pltpu.VMEM((CHUNK,), jnp.int32),
        pltpu.VMEM((CHUNK, H), jnp.float32),
        pltpu.SemaphoreType.DMA,
    ],
)

BA = 400
EBLK = BA * NNN
GRID = ASPLIT // BA
GRID_FULL = B // BA

_dot = functools.partial(jnp.dot, preferred_element_type=jnp.float32)


def _proj_body(atom_ref, w1_ref, b1_ref, w2_ref, a1_ref, a2_ref):
    a = atom_ref[...]
    a1_ref[...] = _dot(a, w1_ref[...]) + b1_ref[...]
    a2_ref[...] = _dot(a, w2_ref[...])


def _edge_atom_stage(m, g, p1, ah, wau1, wau2, bau, t_ref):
    t = jnp.tanh((m + g).reshape(BA, NNN, H) + p1[:, None, :])
    if t_ref is not None:
        t_ref[...] = t.reshape(EBLK, H).astype(t_ref.dtype)
    mean = jnp.sum(t, axis=1) * (1.0 / NNN)
    return jnp.maximum(_dot(mean, wau1) + _dot(ah, wau2) + bau, 0.0)


def _layer0_body(bonds_ref, g_ref, a1_ref, atom_ref, w3_ref, wae1_ref,
                 wae2_ref, bae_ref, wb1_ref, bb_ref, wb2_ref,
                 bh_ref, ah_ref, p1_ref, p2_ref):
    m = _dot(bonds_ref[...].reshape(EBLK, NBF), w3_ref[...])
    ah = _edge_atom_stage(m, g_ref[...], a1_ref[...], atom_ref[...],
                          wae1_ref[...], wae2_ref[...], bae_ref[...], bh_ref)
    ah_ref[...] = ah
    p1_ref[...] = _dot(ah, wb1_ref[...]) + bb_ref[...]
    p2_ref[...] = _dot(ah, wb2_ref[...])


def _conv_body(bhin_ref, g_ref, p1in_ref, ahin_ref, w3_ref, wau1_ref,
               wau2_ref, bau_ref, wb1_ref, bb_ref, wb2_ref,
               bh_ref, ah_ref, p1_ref, p2_ref):
    m = _dot(bhin_ref[...], w3_ref[...])
    ah = _edge_atom_stage(m, g_ref[...], p1in_ref[...], ahin_ref[...],
                          wau1_ref[...], wau2_ref[...], bau_ref[...], bh_ref)
    ah_ref[...] = ah
    p1_ref[...] = _dot(ah, wb1_ref[...]) + bb_ref[...]
    p2_ref[...] = _dot(ah, wb2_ref[...])


def _final_body(bhin_ref, g_ref, p1in_ref, ahin_ref, w3_ref, wau1_ref,
                wau2_ref, bau_ref, wfc_ref, bfc_ref, y_ref):
    m = _dot(bhin_ref[...], w3_ref[...])
    ah = _edge_atom_stage(m, g_ref[...], p1in_ref[...], ahin_ref[...],
                          wau1_ref[...], wau2_ref[...], bau_ref[...], None)
    z = _dot(ah, wfc_ref[...]) + bfc_ref[...]
    y_ref[...] = jnp.maximum(z, 0.0) + jnp.log1p(jnp.exp(-jnp.abs(z)))


def _espec(off=0):
    return pl.BlockSpec((EBLK, H), lambda i, o=off: (o * GRID + i, 0))


def _aspec(off=0, width=H):
    return pl.BlockSpec((BA, width), lambda i, o=off: (o * GRID + i, 0))


def _wspec(rows=H):
    return pl.BlockSpec((rows, H), lambda i: (0, 0))


def _bspec():
    return pl.BlockSpec((1, H), lambda i: (0, 0))


_params = pltpu.CompilerParams(dimension_semantics=("parallel",))


def _pc(body, grid, in_specs, out_specs, out_shapes):
    return pl.pallas_call(
        body,
        grid=(grid,),
        in_specs=in_specs,
        out_specs=out_specs,
        out_shape=out_shapes,
        compiler_params=_params,
    )


def kernel(gmap, atom, bonds, W_be, b_be, W_ae, b_ae, W_bu, b_bu, W_au, b_au,
           W_fc, b_fc):
    idx = gmap.astype(jnp.int32).reshape(E)

    wbe1, wbe2, wbe3 = W_be[:H], W_be[H:2 * H], W_be[2 * H:]
    wae1, wae2 = W_ae[:H], W_ae[H:]
    wbu1, wbu2 = W_bu[:H], W_bu[H:2 * H]
    wbu3 = W_bu[2 * H:].astype(jnp.bfloat16)
    wau1, wau2 = W_au[:H], W_au[H:]
    b_be2 = b_be.reshape(1, H)
    b_ae2 = b_ae.reshape(1, H)
    b_bu2 = b_bu.reshape(1, H)
    b_au2 = b_au.reshape(1, H)
    b_fc2 = b_fc.reshape(1, 1)

    atom_out = jax.ShapeDtypeStruct((ASPLIT, H), jnp.float32)
    bh_out = jax.ShapeDtypeStruct((ESPLIT, H), jnp.bfloat16)

    a1, table = _pc(
        _proj_body, GRID_FULL,
        [_aspec(), _wspec(), _bspec(), _wspec()],
        [_aspec(), _aspec()],
        [jax.ShapeDtypeStruct((B, H), jnp.float32)] * 2,
    )(atom, wbe1, b_be2, wbe2)

    idx_s = [lax.slice_in_dim(idx, s * ESPLIT, (s + 1) * ESPLIT)
             for s in range(NSPLIT)]

    bh_s, ah_s, p1_s, p2_s = [], [], [], []
    for s in range(NSPLIT):
        g = _sc_gather(table, idx_s[s])
        bh, ah, p1, p2 = _pc(
            _layer0_body, GRID,
            [pl.BlockSpec((BA, NNN, NBF), lambda i, o=s: (o * GRID + i, 0, 0)),
             _espec(), _aspec(s), _aspec(s), _wspec(NBF),
             _wspec(), _wspec(), _bspec(), _wspec(), _bspec(), _wspec()],
            [_espec(), _aspec(), _aspec(), _aspec()],
            [bh_out, atom_out, atom_out, atom_out],
        )(bonds, g, a1, atom, wbe3, wae1, wae2, b_ae2, wbu1, b_bu2, wbu2)
        bh_s.append(bh); ah_s.append(ah); p1_s.append(p1); p2_s.append(p2)

    for layer in range(3):
        table = jnp.concatenate(p2_s, axis=0)
        last = layer == 2
        new = [[], [], [], []]
        for s in range(NSPLIT):
            g = _sc_gather(table, idx_s[s])
            if last:
                (y,) = _pc(
                    _final_body, GRID,
                    [_espec(), _espec(), _aspec(), _aspec(),
                     _wspec(), _wspec(), _wspec(), _bspec(),
                     pl.BlockSpec((H, 1), lambda i: (0, 0)),
                     pl.BlockSpec((1, 1), lambda i: (0, 0))],
                    [_aspec(width=1)],
                    [jax.ShapeDtypeStruct((ASPLIT, 1), jnp.float32)],
                )(bh_s[s], g, p1_s[s], ah_s[s], wbu3, wau1, wau2, b_au2,
                  W_fc, b_fc2)
                new[0].append(y)
            else:
                bh, ah, p1, p2 = _pc(
                    _conv_body, GRID,
                    [_espec(), _espec(), _aspec(), _aspec(),
                     _wspec(), _wspec(), _wspec(), _bspec(), _wspec(),
                     _bspec(), _wspec()],
                    [_espec(), _aspec(), _aspec(), _aspec()],
                    [bh_out, atom_out, atom_out, atom_out],
                )(bh_s[s], g, p1_s[s], ah_s[s], wbu3, wau1, wau2, b_au2,
                  wbu1, b_bu2, wbu2)
                new[0].append(bh); new[1].append(ah)
                new[2].append(p1); new[3].append(p2)
        if last:
            return jnp.concatenate(new[0], axis=0)
        bh_s, ah_s, p1_s, p2_s = new


# speedup vs baseline: 3.1332x; 1.1606x over previous
import functools

import jax
import jax.numpy as jnp
from jax import lax
from jax.experimental import pallas as pl
from jax.experimental.pallas import tpu as pltpu
from jax.experimental.pallas import tpu_sc as plsc

B = 10000
NNN = 32
E = B * NNN
NBF = 16
H = 128

NSPLIT = 5
ASPLIT = B // NSPLIT
ESPLIT = ASPLIT * NNN

NC = 2
NS = 16
NW = NC * NS
CHUNK = 128
NCHUNKS = ESPLIT // CHUNK
NB = NCHUNKS // NW
EXTRA = NCHUNKS - NB * NW


def _gather_body(table_hbm, idx_hbm, out_hbm, idx_all, rows0, rows1,
                 sg0, sg1, so0, so1):
    wid = lax.axis_index("s") * NC + lax.axis_index("c")
    base = wid * NB
    pltpu.sync_copy(idx_hbm.at[pl.ds(base * CHUNK, NB * CHUNK)], idx_all)

    rows = (rows0, rows1)
    sg = (sg0, sg1)
    so = (so0, so1)

    def idx_ref(k):
        return idx_all.at[pl.ds(k * CHUNK, CHUNK)]

    def out_ref(k):
        return out_hbm.at[pl.ds((base + k) * CHUNK, CHUNK)]

    def g_start(k, q):
        pltpu.async_copy(table_hbm.at[idx_ref(k)], rows[q], sg[q])

    def g_wait(k, q):
        pltpu.make_async_copy(table_hbm.at[idx_ref(k)], rows[q], sg[q]).wait()

    def o_start(k, q):
        pltpu.async_copy(rows[q], out_ref(k), so[q])

    def o_wait(k, q):
        pltpu.make_async_copy(rows[q], out_ref(k), so[q]).wait()

    g_start(0, 0)
    for k in range(NB):
        p = k % 2
        q = 1 - p
        if k + 1 < NB:
            if k >= 1:
                o_wait(k - 1, q)
            g_start(k + 1, q)
        g_wait(k, p)
        o_start(k, p)
    o_wait(NB - 1, (NB - 1) % 2)

    @pl.when(wid < EXTRA)
    def _():
        c = NB * NW + wid
        pltpu.sync_copy(idx_hbm.at[pl.ds(c * CHUNK, CHUNK)],
                        idx_all.at[pl.ds(0, CHUNK)])
        pltpu.async_copy(table_hbm.at[idx_all.at[pl.ds(0, CHUNK)]],
                         rows0, sg0).wait()
        pltpu.sync_copy(rows0, out_hbm.at[pl.ds(c * CHUNK, CHUNK)])


_sc_gather = pl.kernel(
    _gather_body,
    out_type=jax.ShapeDtypeStruct((ESPLIT, H), jnp.float32),
    mesh=plsc.VectorSubcoreMesh(
        core_axis_name="c", subcore_axis_name="s", num_cores=NC, num_subcores=NS
    ),
    scratch_types=[
        pltpu.VMEM((NB * CHUNK,), jnp.int32),
        pltpu.VMEM((CHUNK, H), jnp.float32),
        pltpu.VMEM((CHUNK, H), jnp.float32),
        pltpu.SemaphoreType.DMA,
        pltpu.SemaphoreType.DMA,
        pltpu.SemaphoreType.DMA,
        pltpu.SemaphoreType.DMA,
    ],
)

BA = 400
EBLK = BA * NNN
GRID = ASPLIT // BA
GRID_FULL = B // BA

_dot = functools.partial(jnp.dot, preferred_element_type=jnp.float32)


def _proj_body(atom_ref, w1_ref, b1_ref, w2_ref, a1_ref, a2_ref):
    a = atom_ref[...]
    a1_ref[...] = _dot(a, w1_ref[...]) + b1_ref[...]
    a2_ref[...] = _dot(a, w2_ref[...])


def _edge_atom_stage(m, g, p1, ah, wau1, wau2, bau, t_ref):
    t = jnp.tanh((m + g).reshape(BA, NNN, H) + p1[:, None, :])
    if t_ref is not None:
        t_ref[...] = t.reshape(EBLK, H).astype(t_ref.dtype)
    mean = jnp.sum(t, axis=1) * (1.0 / NNN)
    return jnp.maximum(_dot(mean, wau1) + _dot(ah, wau2) + bau, 0.0)


def _layer0_body(bonds_ref, g_ref, a1_ref, atom_ref, w3_ref, wae1_ref,
                 wae2_ref, bae_ref, wb1_ref, bb_ref, wb2_ref,
                 bh_ref, ah_ref, p1_ref, p2_ref):
    m = _dot(bonds_ref[...].reshape(EBLK, NBF), w3_ref[...])
    ah = _edge_atom_stage(m, g_ref[...], a1_ref[...], atom_ref[...],
                          wae1_ref[...], wae2_ref[...], bae_ref[...], bh_ref)
    ah_ref[...] = ah
    p1_ref[...] = _dot(ah, wb1_ref[...]) + bb_ref[...]
    p2_ref[...] = _dot(ah, wb2_ref[...])


def _conv_body(bhin_ref, g_ref, p1in_ref, ahin_ref, w3_ref, wau1_ref,
               wau2_ref, bau_ref, wb1_ref, bb_ref, wb2_ref,
               bh_ref, ah_ref, p1_ref, p2_ref):
    m = _dot(bhin_ref[...], w3_ref[...])
    ah = _edge_atom_stage(m, g_ref[...], p1in_ref[...], ahin_ref[...],
                          wau1_ref[...], wau2_ref[...], bau_ref[...], bh_ref)
    ah_ref[...] = ah
    p1_ref[...] = _dot(ah, wb1_ref[...]) + bb_ref[...]
    p2_ref[...] = _dot(ah, wb2_ref[...])


def _final_body(bhin_ref, g_ref, p1in_ref, ahin_ref, w3_ref, wau1_ref,
                wau2_ref, bau_ref, wfc_ref, bfc_ref, y_ref):
    m = _dot(bhin_ref[...], w3_ref[...])
    ah = _edge_atom_stage(m, g_ref[...], p1in_ref[...], ahin_ref[...],
                          wau1_ref[...], wau2_ref[...], bau_ref[...], None)
    z = _dot(ah, wfc_ref[...]) + bfc_ref[...]
    y_ref[...] = jnp.maximum(z, 0.0) + jnp.log1p(jnp.exp(-jnp.abs(z)))


def _espec(off=0):
    return pl.BlockSpec((EBLK, H), lambda i, o=off: (o * GRID + i, 0))


def _aspec(off=0, width=H):
    return pl.BlockSpec((BA, width), lambda i, o=off: (o * GRID + i, 0))


def _wspec(rows=H):
    return pl.BlockSpec((rows, H), lambda i: (0, 0))


def _bspec():
    return pl.BlockSpec((1, H), lambda i: (0, 0))


_params = pltpu.CompilerParams(dimension_semantics=("parallel",))


def _pc(body, grid, in_specs, out_specs, out_shapes):
    return pl.pallas_call(
        body,
        grid=(grid,),
        in_specs=in_specs,
        out_specs=out_specs,
        out_shape=out_shapes,
        compiler_params=_params,
    )


def kernel(gmap, atom, bonds, W_be, b_be, W_ae, b_ae, W_bu, b_bu, W_au, b_au,
           W_fc, b_fc):
    idx = gmap.astype(jnp.int32).reshape(E)

    wbe1, wbe2, wbe3 = W_be[:H], W_be[H:2 * H], W_be[2 * H:]
    wae1, wae2 = W_ae[:H], W_ae[H:]
    wbu1, wbu2 = W_bu[:H], W_bu[H:2 * H]
    wbu3 = W_bu[2 * H:].astype(jnp.bfloat16)
    wau1, wau2 = W_au[:H], W_au[H:]
    b_be2 = b_be.reshape(1, H)
    b_ae2 = b_ae.reshape(1, H)
    b_bu2 = b_bu.reshape(1, H)
    b_au2 = b_au.reshape(1, H)
    b_fc2 = b_fc.reshape(1, 1)

    atom_out = jax.ShapeDtypeStruct((ASPLIT, H), jnp.float32)
    bh_out = jax.ShapeDtypeStruct((ESPLIT, H), jnp.bfloat16)

    a1, table = _pc(
        _proj_body, GRID_FULL,
        [_aspec(), _wspec(), _bspec(), _wspec()],
        [_aspec(), _aspec()],
        [jax.ShapeDtypeStruct((B, H), jnp.float32)] * 2,
    )(atom, wbe1, b_be2, wbe2)

    idx_s = [lax.slice_in_dim(idx, s * ESPLIT, (s + 1) * ESPLIT)
             for s in range(NSPLIT)]

    bh_s, ah_s, p1_s, p2_s = [], [], [], []
    for s in range(NSPLIT):
        g = _sc_gather(table, idx_s[s])
        bh, ah, p1, p2 = _pc(
            _layer0_body, GRID,
            [pl.BlockSpec((BA, NNN, NBF), lambda i, o=s: (o * GRID + i, 0, 0)),
             _espec(), _aspec(s), _aspec(s), _wspec(NBF),
             _wspec(), _wspec(), _bspec(), _wspec(), _bspec(), _wspec()],
            [_espec(), _aspec(), _aspec(), _aspec()],
            [bh_out, atom_out, atom_out, atom_out],
        )(bonds, g, a1, atom, wbe3, wae1, wae2, b_ae2, wbu1, b_bu2, wbu2)
        bh_s.append(bh); ah_s.append(ah); p1_s.append(p1); p2_s.append(p2)

    for layer in range(3):
        table = jnp.concatenate(p2_s, axis=0)
        last = layer == 2
        new = [[], [], [], []]
        for s in range(NSPLIT):
            g = _sc_gather(table, idx_s[s])
            if last:
                (y,) = _pc(
                    _final_body, GRID,
                    [_espec(), _espec(), _aspec(), _aspec(),
                     _wspec(), _wspec(), _wspec(), _bspec(),
                     pl.BlockSpec((H, 1), lambda i: (0, 0)),
                     pl.BlockSpec((1, 1), lambda i: (0, 0))],
                    [_aspec(width=1)],
                    [jax.ShapeDtypeStruct((ASPLIT, 1), jnp.float32)],
                )(bh_s[s], g, p1_s[s], ah_s[s], wbu3, wau1, wau2, b_au2,
                  W_fc, b_fc2)
                new[0].append(y)
            else:
                bh, ah, p1, p2 = _pc(
                    _conv_body, GRID,
                    [_espec(), _espec(), _aspec(), _aspec(),
                     _wspec(), _wspec(), _wspec(), _bspec(), _wspec(),
                     _bspec(), _wspec()],
                    [_espec(), _aspec(), _aspec(), _aspec()],
                    [bh_out, atom_out, atom_out, atom_out],
                )(bh_s[s], g, p1_s[s], ah_s[s], wbu3, wau1, wau2, b_au2,
                  wbu1, b_bu2, wbu2)
                new[0].append(bh); new[1].append(ah)
                new[2].append(p1); new[3].append(p2)
        if last:
            return jnp.concatenate(new[0], axis=0)
        bh_s, ah_s, p1_s, p2_s = new


# speedup vs baseline: 3.1586x; 1.0081x over previous
import functools

import jax
import jax.numpy as jnp
from jax import lax
from jax.experimental import pallas as pl
from jax.experimental.pallas import tpu as pltpu
from jax.experimental.pallas import tpu_sc as plsc

B = 10000
NNN = 32
E = B * NNN
NBF = 16
H = 128

NSPLIT = 5
ASPLIT = B // NSPLIT
ESPLIT = ASPLIT * NNN

NC = 2
NS = 16
NW = NC * NS
CHUNK = 128
NCHUNKS = ESPLIT // CHUNK
NB = NCHUNKS // NW
EXTRA = NCHUNKS - NB * NW


def _gather_body(table_hbm, idx_hbm, out_hbm, idx_all, rows0, rows1,
                 sg0, sg1, so0, so1):
    wid = lax.axis_index("s") * NC + lax.axis_index("c")
    base = wid * NB
    pltpu.sync_copy(idx_hbm.at[pl.ds(base * CHUNK, NB * CHUNK)], idx_all)

    rows = (rows0, rows1)
    sg = (sg0, sg1)
    so = (so0, so1)

    def idx_ref(k):
        return idx_all.at[pl.ds(k * CHUNK, CHUNK)]

    def out_ref(k):
        return out_hbm.at[pl.ds((base + k) * CHUNK, CHUNK)]

    def g_start(k, q):
        pltpu.async_copy(table_hbm.at[idx_ref(k)], rows[q], sg[q])

    def g_wait(k, q):
        pltpu.make_async_copy(table_hbm.at[idx_ref(k)], rows[q], sg[q]).wait()

    def o_start(k, q):
        pltpu.async_copy(rows[q], out_ref(k), so[q])

    def o_wait(k, q):
        pltpu.make_async_copy(rows[q], out_ref(k), so[q]).wait()

    g_start(0, 0)
    for k in range(NB):
        p = k % 2
        q = 1 - p
        if k + 1 < NB:
            if k >= 1:
                o_wait(k - 1, q)
            g_start(k + 1, q)
        g_wait(k, p)
        o_start(k, p)
    if NB >= 2:
        o_wait(NB - 2, (NB - 2) % 2)
    o_wait(NB - 1, (NB - 1) % 2)

    @pl.when(wid < EXTRA)
    def _():
        c = NB * NW + wid
        pltpu.sync_copy(idx_hbm.at[pl.ds(c * CHUNK, CHUNK)],
                        idx_all.at[pl.ds(0, CHUNK)])
        pltpu.async_copy(table_hbm.at[idx_all.at[pl.ds(0, CHUNK)]],
                         rows0, sg0).wait()
        pltpu.sync_copy(rows0, out_hbm.at[pl.ds(c * CHUNK, CHUNK)])


_sc_gather = pl.kernel(
    _gather_body,
    out_type=jax.ShapeDtypeStruct((ESPLIT, H), jnp.float32),
    mesh=plsc.VectorSubcoreMesh(
        core_axis_name="c", subcore_axis_name="s", num_cores=NC, num_subcores=NS
    ),
    scratch_types=[
        pltpu.VMEM((NB * CHUNK,), jnp.int32),
        pltpu.VMEM((CHUNK, H), jnp.float32),
        pltpu.VMEM((CHUNK, H), jnp.float32),
        pltpu.SemaphoreType.DMA,
        pltpu.SemaphoreType.DMA,
        pltpu.SemaphoreType.DMA,
        pltpu.SemaphoreType.DMA,
    ],
)

BA = 400
EBLK = BA * NNN
GRID = ASPLIT // BA
GRID_FULL = B // BA

_dot = functools.partial(jnp.dot, preferred_element_type=jnp.float32)


def _proj_body(atom_ref, w1_ref, b1_ref, w2_ref, a1_ref, a2_ref):
    a = atom_ref[...]
    a1_ref[...] = _dot(a, w1_ref[...]) + b1_ref[...]
    a2_ref[...] = _dot(a, w2_ref[...])


def _edge_atom_stage(m, g, p1, ah, wau1, wau2, bau, t_ref):
    t = jnp.tanh((m + g).reshape(BA, NNN, H) + p1[:, None, :])
    if t_ref is not None:
        t_ref[...] = t.reshape(EBLK, H).astype(t_ref.dtype)
    mean = jnp.sum(t, axis=1) * (1.0 / NNN)
    return jnp.maximum(_dot(mean, wau1) + _dot(ah, wau2) + bau, 0.0)


def _layer0_body(bonds_ref, g_ref, a1_ref, atom_ref, w3_ref, wae1_ref,
                 wae2_ref, bae_ref, wb1_ref, bb_ref, wb2_ref,
                 bh_ref, ah_ref, p1_ref, p2_ref):
    m = _dot(bonds_ref[...].reshape(EBLK, NBF), w3_ref[...])
    ah = _edge_atom_stage(m, g_ref[...], a1_ref[...], atom_ref[...],
                          wae1_ref[...], wae2_ref[...], bae_ref[...], bh_ref)
    ah_ref[...] = ah
    p1_ref[...] = _dot(ah, wb1_ref[...]) + bb_ref[...]
    p2_ref[...] = _dot(ah, wb2_ref[...])


def _conv_body(bhin_ref, g_ref, p1in_ref, ahin_ref, w3_ref, wau1_ref,
               wau2_ref, bau_ref, wb1_ref, bb_ref, wb2_ref,
               bh_ref, ah_ref, p1_ref, p2_ref):
    m = _dot(bhin_ref[...], w3_ref[...])
    ah = _edge_atom_stage(m, g_ref[...], p1in_ref[...], ahin_ref[...],
                          wau1_ref[...], wau2_ref[...], bau_ref[...], bh_ref)
    ah_ref[...] = ah
    p1_ref[...] = _dot(ah, wb1_ref[...]) + bb_ref[...]
    p2_ref[...] = _dot(ah, wb2_ref[...])


def _final_body(bhin_ref, g_ref, p1in_ref, ahin_ref, w3_ref, wau1_ref,
                wau2_ref, bau_ref, wfc_ref, bfc_ref, y_ref):
    m = _dot(bhin_ref[...], w3_ref[...])
    ah = _edge_atom_stage(m, g_ref[...], p1in_ref[...], ahin_ref[...],
                          wau1_ref[...], wau2_ref[...], bau_ref[...], None)
    z = _dot(ah, wfc_ref[...]) + bfc_ref[...]
    y_ref[...] = jnp.maximum(z, 0.0) + jnp.log1p(jnp.exp(-jnp.abs(z)))


def _espec(off=0):
    return pl.BlockSpec((EBLK, H), lambda i, o=off: (o * GRID + i, 0))


def _aspec(off=0, width=H):
    return pl.BlockSpec((BA, width), lambda i, o=off: (o * GRID + i, 0))


def _wspec(rows=H):
    return pl.BlockSpec((rows, H), lambda i: (0, 0))


def _bspec():
    return pl.BlockSpec((1, H), lambda i: (0, 0))


_params = pltpu.CompilerParams(dimension_semantics=("parallel",))


def _pc(body, grid, in_specs, out_specs, out_shapes):
    return pl.pallas_call(
        body,
        grid=(grid,),
        in_specs=in_specs,
        out_specs=out_specs,
        out_shape=out_shapes,
        compiler_params=_params,
    )


def kernel(gmap, atom, bonds, W_be, b_be, W_ae, b_ae, W_bu, b_bu, W_au, b_au,
           W_fc, b_fc):
    idx = gmap.astype(jnp.int32).reshape(E)

    wbe1, wbe2, wbe3 = W_be[:H], W_be[H:2 * H], W_be[2 * H:]
    wae1, wae2 = W_ae[:H], W_ae[H:]
    wbu1, wbu2 = W_bu[:H], W_bu[H:2 * H]
    wbu3 = W_bu[2 * H:].astype(jnp.bfloat16)
    wau1, wau2 = W_au[:H], W_au[H:]
    b_be2 = b_be.reshape(1, H)
    b_ae2 = b_ae.reshape(1, H)
    b_bu2 = b_bu.reshape(1, H)
    b_au2 = b_au.reshape(1, H)
    b_fc2 = b_fc.reshape(1, 1)

    atom_out = jax.ShapeDtypeStruct((ASPLIT, H), jnp.float32)
    bh_out = jax.ShapeDtypeStruct((ESPLIT, H), jnp.bfloat16)

    a1, table = _pc(
        _proj_body, GRID_FULL,
        [_aspec(), _wspec(), _bspec(), _wspec()],
        [_aspec(), _aspec()],
        [jax.ShapeDtypeStruct((B, H), jnp.float32)] * 2,
    )(atom, wbe1, b_be2, wbe2)

    idx_s = [lax.slice_in_dim(idx, s * ESPLIT, (s + 1) * ESPLIT)
             for s in range(NSPLIT)]

    bh_s, ah_s, p1_s, p2_s = [], [], [], []
    for s in range(NSPLIT):
        g = _sc_gather(table, idx_s[s])
        bh, ah, p1, p2 = _pc(
            _layer0_body, GRID,
            [pl.BlockSpec((BA, NNN, NBF), lambda i, o=s: (o * GRID + i, 0, 0)),
             _espec(), _aspec(s), _aspec(s), _wspec(NBF),
             _wspec(), _wspec(), _bspec(), _wspec(), _bspec(), _wspec()],
            [_espec(), _aspec(), _aspec(), _aspec()],
            [bh_out, atom_out, atom_out, atom_out],
        )(bonds, g, a1, atom, wbe3, wae1, wae2, b_ae2, wbu1, b_bu2, wbu2)
        bh_s.append(bh); ah_s.append(ah); p1_s.append(p1); p2_s.append(p2)

    for layer in range(3):
        table = jnp.concatenate(p2_s, axis=0)
        last = layer == 2
        new = [[], [], [], []]
        for s in range(NSPLIT):
            g = _sc_gather(table, idx_s[s])
            if last:
                (y,) = _pc(
                    _final_body, GRID,
                    [_espec(), _espec(), _aspec(), _aspec(),
                     _wspec(), _wspec(), _wspec(), _bspec(),
                     pl.BlockSpec((H, 1), lambda i: (0, 0)),
                     pl.BlockSpec((1, 1), lambda i: (0, 0))],
                    [_aspec(width=1)],
                    [jax.ShapeDtypeStruct((ASPLIT, 1), jnp.float32)],
                )(bh_s[s], g, p1_s[s], ah_s[s], wbu3, wau1, wau2, b_au2,
                  W_fc, b_fc2)
                new[0].append(y)
            else:
                bh, ah, p1, p2 = _pc(
                    _conv_body, GRID,
                    [_espec(), _espec(), _aspec(), _aspec(),
                     _wspec(), _wspec(), _wspec(), _bspec(), _wspec(),
                     _bspec(), _wspec()],
                    [_espec(), _aspec(), _aspec(), _aspec()],
                    [bh_out, atom_out, atom_out, atom_out],
                )(bh_s[s], g, p1_s[s], ah_s[s], wbu3, wau1, wau2, b_au2,
                  wbu1, b_bu2, wbu2)
                new[0].append(bh); new[1].append(ah)
                new[2].append(p1); new[3].append(p2)
        if last:
            return jnp.concatenate(new[0], axis=0)
        bh_s, ah_s, p1_s, p2_s = new


# speedup vs baseline: 3.1634x; 1.0015x over previous
import functools

import jax
import jax.numpy as jnp
from jax import lax
from jax.experimental import pallas as pl
from jax.experimental.pallas import tpu as pltpu
from jax.experimental.pallas import tpu_sc as plsc

B = 10000
NNN = 32
E = B * NNN
NBF = 16
H = 128

NSPLIT = 5
ASPLIT = B // NSPLIT
ESPLIT = ASPLIT * NNN

NC = 2
NS = 16
NW = NC * NS
CHUNK = 128
SPAN = ESPLIT // NW
CS = [CHUNK] * (SPAN // CHUNK) + ([SPAN % CHUNK] if SPAN % CHUNK else [])
OFF = [sum(CS[:k]) for k in range(len(CS))]
NB = len(CS)
NBUF = 4
LOOKAHEAD = NBUF - 1


def _gather_body(table_hbm, idx_hbm, out_hbm, idx_all, *bufs):
    rows = bufs[:NBUF]
    sg = bufs[NBUF:2 * NBUF]
    so = bufs[2 * NBUF:]
    wid = lax.axis_index("s") * NC + lax.axis_index("c")
    base = wid * SPAN
    pltpu.sync_copy(idx_hbm.at[pl.ds(base, SPAN)], idx_all)

    def idx_ref(k):
        return idx_all.at[pl.ds(OFF[k], CS[k])]

    def rows_ref(k, q):
        return rows[q] if CS[k] == CHUNK else rows[q].at[pl.ds(0, CS[k])]

    def out_ref(k):
        return out_hbm.at[pl.ds(base + OFF[k], CS[k])]

    def g_start(k, q):
        pltpu.async_copy(table_hbm.at[idx_ref(k)], rows_ref(k, q), sg[q])

    def g_wait(k, q):
        pltpu.make_async_copy(table_hbm.at[idx_ref(k)], rows_ref(k, q),
                              sg[q]).wait()

    def o_start(k, q):
        pltpu.async_copy(rows_ref(k, q), out_ref(k), so[q])

    def o_wait(k, q):
        pltpu.make_async_copy(rows_ref(k, q), out_ref(k), so[q]).wait()

    for j in range(min(LOOKAHEAD, NB)):
        g_start(j, j % NBUF)
    for k in range(NB):
        kk = k + LOOKAHEAD
        if kk < NB:
            q = kk % NBUF
            if kk >= NBUF:
                o_wait(kk - NBUF, q)
            g_start(kk, q)
        p = k % NBUF
        g_wait(k, p)
        o_start(k, p)
    for k in range(max(0, NB - NBUF), NB):
        o_wait(k, k % NBUF)


_sc_gather = pl.kernel(
    _gather_body,
    out_type=jax.ShapeDtypeStruct((ESPLIT, H), jnp.float32),
    mesh=plsc.VectorSubcoreMesh(
        core_axis_name="c", subcore_axis_name="s", num_cores=NC, num_subcores=NS
    ),
    scratch_types=[pltpu.VMEM((SPAN,), jnp.int32)]
    + [pltpu.VMEM((CHUNK, H), jnp.float32)] * NBUF
    + [pltpu.SemaphoreType.DMA] * (2 * NBUF),
)

BA = 400
EBLK = BA * NNN
GRID = ASPLIT // BA
GRID_FULL = B // BA

_dot = functools.partial(jnp.dot, preferred_element_type=jnp.float32)


def _proj_body(atom_ref, w1_ref, b1_ref, w2_ref, a1_ref, a2_ref):
    a = atom_ref[...]
    a1_ref[...] = _dot(a, w1_ref[...]) + b1_ref[...]
    a2_ref[...] = _dot(a, w2_ref[...])


def _edge_atom_stage(m, g, p1, ah, wau1, wau2, bau, t_ref):
    t = jnp.tanh((m + g).reshape(BA, NNN, H) + p1[:, None, :])
    if t_ref is not None:
        t_ref[...] = t.reshape(EBLK, H).astype(t_ref.dtype)
    mean = jnp.sum(t, axis=1) * (1.0 / NNN)
    return jnp.maximum(_dot(mean, wau1) + _dot(ah, wau2) + bau, 0.0)


def _layer0_body(bonds_ref, g_ref, a1_ref, atom_ref, w3_ref, wae1_ref,
                 wae2_ref, bae_ref, wb1_ref, bb_ref, wb2_ref,
                 bh_ref, ah_ref, p1_ref, p2_ref):
    m = _dot(bonds_ref[...].reshape(EBLK, NBF), w3_ref[...])
    ah = _edge_atom_stage(m, g_ref[...], a1_ref[...], atom_ref[...],
                          wae1_ref[...], wae2_ref[...], bae_ref[...], bh_ref)
    ah_ref[...] = ah
    p1_ref[...] = _dot(ah, wb1_ref[...]) + bb_ref[...]
    p2_ref[...] = _dot(ah, wb2_ref[...])


def _conv_body(bhin_ref, g_ref, p1in_ref, ahin_ref, w3_ref, wau1_ref,
               wau2_ref, bau_ref, wb1_ref, bb_ref, wb2_ref,
               bh_ref, ah_ref, p1_ref, p2_ref):
    m = _dot(bhin_ref[...], w3_ref[...])
    ah = _edge_atom_stage(m, g_ref[...], p1in_ref[...], ahin_ref[...],
                          wau1_ref[...], wau2_ref[...], bau_ref[...], bh_ref)
    ah_ref[...] = ah
    p1_ref[...] = _dot(ah, wb1_ref[...]) + bb_ref[...]
    p2_ref[...] = _dot(ah, wb2_ref[...])


def _final_body(bhin_ref, g_ref, p1in_ref, ahin_ref, w3_ref, wau1_ref,
                wau2_ref, bau_ref, wfc_ref, bfc_ref, y_ref):
    m = _dot(bhin_ref[...], w3_ref[...])
    ah = _edge_atom_stage(m, g_ref[...], p1in_ref[...], ahin_ref[...],
                          wau1_ref[...], wau2_ref[...], bau_ref[...], None)
    z = _dot(ah, wfc_ref[...]) + bfc_ref[...]
    y_ref[...] = jnp.maximum(z, 0.0) + jnp.log1p(jnp.exp(-jnp.abs(z)))


def _espec(off=0):
    return pl.BlockSpec((EBLK, H), lambda i, o=off: (o * GRID + i, 0))


def _aspec(off=0, width=H):
    return pl.BlockSpec((BA, width), lambda i, o=off: (o * GRID + i, 0))


def _wspec(rows=H):
    return pl.BlockSpec((rows, H), lambda i: (0, 0))


def _bspec():
    return pl.BlockSpec((1, H), lambda i: (0, 0))


_params = pltpu.CompilerParams(dimension_semantics=("parallel",))


def _pc(body, grid, in_specs, out_specs, out_shapes):
    return pl.pallas_call(
        body,
        grid=(grid,),
        in_specs=in_specs,
        out_specs=out_specs,
        out_shape=out_shapes,
        compiler_params=_params,
    )


def kernel(gmap, atom, bonds, W_be, b_be, W_ae, b_ae, W_bu, b_bu, W_au, b_au,
           W_fc, b_fc):
    idx = gmap.astype(jnp.int32).reshape(E)

    wbe1, wbe2, wbe3 = W_be[:H], W_be[H:2 * H], W_be[2 * H:]
    wae1, wae2 = W_ae[:H], W_ae[H:]
    wbu1, wbu2 = W_bu[:H], W_bu[H:2 * H]
    wbu3 = W_bu[2 * H:].astype(jnp.bfloat16)
    wau1, wau2 = W_au[:H], W_au[H:]
    b_be2 = b_be.reshape(1, H)
    b_ae2 = b_ae.reshape(1, H)
    b_bu2 = b_bu.reshape(1, H)
    b_au2 = b_au.reshape(1, H)
    b_fc2 = b_fc.reshape(1, 1)

    atom_out = jax.ShapeDtypeStruct((ASPLIT, H), jnp.float32)
    bh_out = jax.ShapeDtypeStruct((ESPLIT, H), jnp.bfloat16)

    a1, table = _pc(
        _proj_body, GRID_FULL,
        [_aspec(), _wspec(), _bspec(), _wspec()],
        [_aspec(), _aspec()],
        [jax.ShapeDtypeStruct((B, H), jnp.float32)] * 2,
    )(atom, wbe1, b_be2, wbe2)

    idx_s = [lax.slice_in_dim(idx, s * ESPLIT, (s + 1) * ESPLIT)
             for s in range(NSPLIT)]

    bh_s, ah_s, p1_s, p2_s = [], [], [], []
    for s in range(NSPLIT):
        g = _sc_gather(table, idx_s[s])
        bh, ah, p1, p2 = _pc(
            _layer0_body, GRID,
            [pl.BlockSpec((BA, NNN, NBF), lambda i, o=s: (o * GRID + i, 0, 0)),
             _espec(), _aspec(s), _aspec(s), _wspec(NBF),
             _wspec(), _wspec(), _bspec(), _wspec(), _bspec(), _wspec()],
            [_espec(), _aspec(), _aspec(), _aspec()],
            [bh_out, atom_out, atom_out, atom_out],
        )(bonds, g, a1, atom, wbe3, wae1, wae2, b_ae2, wbu1, b_bu2, wbu2)
        bh_s.append(bh); ah_s.append(ah); p1_s.append(p1); p2_s.append(p2)

    for layer in range(3):
        table = jnp.concatenate(p2_s, axis=0)
        last = layer == 2
        new = [[], [], [], []]
        for s in range(NSPLIT):
            g = _sc_gather(table, idx_s[s])
            if last:
                (y,) = _pc(
                    _final_body, GRID,
                    [_espec(), _espec(), _aspec(), _aspec(),
                     _wspec(), _wspec(), _wspec(), _bspec(),
                     pl.BlockSpec((H, 1), lambda i: (0, 0)),
                     pl.BlockSpec((1, 1), lambda i: (0, 0))],
                    [_aspec(width=1)],
                    [jax.ShapeDtypeStruct((ASPLIT, 1), jnp.float32)],
                )(bh_s[s], g, p1_s[s], ah_s[s], wbu3, wau1, wau2, b_au2,
                  W_fc, b_fc2)
                new[0].append(y)
            else:
                bh, ah, p1, p2 = _pc(
                    _conv_body, GRID,
                    [_espec(), _espec(), _aspec(), _aspec(),
                     _wspec(), _wspec(), _wspec(), _bspec(), _wspec(),
                     _bspec(), _wspec()],
                    [_espec(), _aspec(), _aspec(), _aspec()],
                    [bh_out, atom_out, atom_out, atom_out],
                )(bh_s[s], g, p1_s[s], ah_s[s], wbu3, wau1, wau2, b_au2,
                  wbu1, b_bu2, wbu2)
                new[0].append(bh); new[1].append(ah)
                new[2].append(p1); new[3].append(p2)
        if last:
            return jnp.concatenate(new[0], axis=0)
        bh_s, ah_s, p1_s, p2_s = new


# speedup vs baseline: 3.8763x; 1.2254x over previous
import functools

import jax
import jax.numpy as jnp
from jax import lax
from jax.experimental import pallas as pl
from jax.experimental.pallas import tpu as pltpu
from jax.experimental.pallas import tpu_sc as plsc

B = 10000
NNN = 32
E = B * NNN
NBF = 16
H = 128

NSPLIT = 5
ASPLIT = B // NSPLIT
ESPLIT = ASPLIT * NNN

NC = 2
NS = 16
NW = NC * NS
CHUNK = 128
SPAN = ESPLIT // NW
CS = [CHUNK] * (SPAN // CHUNK) + ([SPAN % CHUNK] if SPAN % CHUNK else [])
OFF = [sum(CS[:k]) for k in range(len(CS))]
NB = len(CS)
NBUF = 2
LOOKAHEAD = NBUF - 1


def _gather_body(table_hbm, idx_hbm, out_hbm, idx_all, tbl, *bufs):
    rows = bufs[:NBUF]
    sg = bufs[NBUF:2 * NBUF]
    so = bufs[2 * NBUF:]
    wid = lax.axis_index("s") * NC + lax.axis_index("c")
    base = wid * SPAN

    @pl.when(lax.axis_index("s") == 0)
    def _():
        pltpu.sync_copy(table_hbm, tbl)

    pltpu.sync_copy(idx_hbm.at[pl.ds(base, SPAN)], idx_all)
    plsc.subcore_barrier()

    def idx_ref(k):
        return idx_all.at[pl.ds(OFF[k], CS[k])]

    def rows_ref(k, q):
        return rows[q] if CS[k] == CHUNK else rows[q].at[pl.ds(0, CS[k])]

    def out_ref(k):
        return out_hbm.at[pl.ds(base + OFF[k], CS[k])]

    def g_start(k, q):
        pltpu.async_copy(tbl.at[idx_ref(k)], rows_ref(k, q), sg[q])

    def g_wait(k, q):
        pltpu.make_async_copy(tbl.at[idx_ref(k)], rows_ref(k, q),
                              sg[q]).wait()

    def o_start(k, q):
        pltpu.async_copy(rows_ref(k, q), out_ref(k), so[q])

    def o_wait(k, q):
        pltpu.make_async_copy(rows_ref(k, q), out_ref(k), so[q]).wait()

    for j in range(min(LOOKAHEAD, NB)):
        g_start(j, j % NBUF)
    for k in range(NB):
        kk = k + LOOKAHEAD
        if kk < NB:
            q = kk % NBUF
            if kk >= NBUF:
                o_wait(kk - NBUF, q)
            g_start(kk, q)
        p = k % NBUF
        g_wait(k, p)
        o_start(k, p)
    for k in range(max(0, NB - NBUF), NB):
        o_wait(k, k % NBUF)


_sc_gather = pl.kernel(
    _gather_body,
    out_type=jax.ShapeDtypeStruct((ESPLIT, H), jnp.float32),
    mesh=plsc.VectorSubcoreMesh(
        core_axis_name="c", subcore_axis_name="s", num_cores=NC, num_subcores=NS
    ),
    scratch_types=[pltpu.VMEM((SPAN,), jnp.int32),
                   pltpu.VMEM_SHARED((B, H), jnp.float32)]
    + [pltpu.VMEM((CHUNK, H), jnp.float32)] * NBUF
    + [pltpu.SemaphoreType.DMA] * (2 * NBUF),
)

BA = 400
EBLK = BA * NNN
GRID = ASPLIT // BA
GRID_FULL = B // BA

_dot = functools.partial(jnp.dot, preferred_element_type=jnp.float32)


def _proj_body(atom_ref, w1_ref, b1_ref, w2_ref, a1_ref, a2_ref):
    a = atom_ref[...]
    a1_ref[...] = _dot(a, w1_ref[...]) + b1_ref[...]
    a2_ref[...] = _dot(a, w2_ref[...])


def _edge_atom_stage(m, g, p1, ah, wau1, wau2, bau, t_ref):
    t = jnp.tanh((m + g).reshape(BA, NNN, H) + p1[:, None, :])
    if t_ref is not None:
        t_ref[...] = t.reshape(EBLK, H).astype(t_ref.dtype)
    mean = jnp.sum(t, axis=1) * (1.0 / NNN)
    return jnp.maximum(_dot(mean, wau1) + _dot(ah, wau2) + bau, 0.0)


def _layer0_body(bonds_ref, g_ref, a1_ref, atom_ref, w3_ref, wae1_ref,
                 wae2_ref, bae_ref, wb1_ref, bb_ref, wb2_ref,
                 bh_ref, ah_ref, p1_ref, p2_ref):
    m = _dot(bonds_ref[...].reshape(EBLK, NBF), w3_ref[...])
    ah = _edge_atom_stage(m, g_ref[...], a1_ref[...], atom_ref[...],
                          wae1_ref[...], wae2_ref[...], bae_ref[...], bh_ref)
    ah_ref[...] = ah
    p1_ref[...] = _dot(ah, wb1_ref[...]) + bb_ref[...]
    p2_ref[...] = _dot(ah, wb2_ref[...])


def _conv_body(bhin_ref, g_ref, p1in_ref, ahin_ref, w3_ref, wau1_ref,
               wau2_ref, bau_ref, wb1_ref, bb_ref, wb2_ref,
               bh_ref, ah_ref, p1_ref, p2_ref):
    m = _dot(bhin_ref[...], w3_ref[...])
    ah = _edge_atom_stage(m, g_ref[...], p1in_ref[...], ahin_ref[...],
                          wau1_ref[...], wau2_ref[...], bau_ref[...], bh_ref)
    ah_ref[...] = ah
    p1_ref[...] = _dot(ah, wb1_ref[...]) + bb_ref[...]
    p2_ref[...] = _dot(ah, wb2_ref[...])


def _final_body(bhin_ref, g_ref, p1in_ref, ahin_ref, w3_ref, wau1_ref,
                wau2_ref, bau_ref, wfc_ref, bfc_ref, y_ref):
    m = _dot(bhin_ref[...], w3_ref[...])
    ah = _edge_atom_stage(m, g_ref[...], p1in_ref[...], ahin_ref[...],
                          wau1_ref[...], wau2_ref[...], bau_ref[...], None)
    z = _dot(ah, wfc_ref[...]) + bfc_ref[...]
    y_ref[...] = jnp.maximum(z, 0.0) + jnp.log1p(jnp.exp(-jnp.abs(z)))


def _espec(off=0):
    return pl.BlockSpec((EBLK, H), lambda i, o=off: (o * GRID + i, 0))


def _aspec(off=0, width=H):
    return pl.BlockSpec((BA, width), lambda i, o=off: (o * GRID + i, 0))


def _wspec(rows=H):
    return pl.BlockSpec((rows, H), lambda i: (0, 0))


def _bspec():
    return pl.BlockSpec((1, H), lambda i: (0, 0))


_params = pltpu.CompilerParams(dimension_semantics=("parallel",))


def _pc(body, grid, in_specs, out_specs, out_shapes):
    return pl.pallas_call(
        body,
        grid=(grid,),
        in_specs=in_specs,
        out_specs=out_specs,
        out_shape=out_shapes,
        compiler_params=_params,
    )


def kernel(gmap, atom, bonds, W_be, b_be, W_ae, b_ae, W_bu, b_bu, W_au, b_au,
           W_fc, b_fc):
    idx = gmap.astype(jnp.int32).reshape(E)

    wbe1, wbe2, wbe3 = W_be[:H], W_be[H:2 * H], W_be[2 * H:]
    wae1, wae2 = W_ae[:H], W_ae[H:]
    wbu1, wbu2 = W_bu[:H], W_bu[H:2 * H]
    wbu3 = W_bu[2 * H:].astype(jnp.bfloat16)
    wau1, wau2 = W_au[:H], W_au[H:]
    b_be2 = b_be.reshape(1, H)
    b_ae2 = b_ae.reshape(1, H)
    b_bu2 = b_bu.reshape(1, H)
    b_au2 = b_au.reshape(1, H)
    b_fc2 = b_fc.reshape(1, 1)

    atom_out = jax.ShapeDtypeStruct((ASPLIT, H), jnp.float32)
    bh_out = jax.ShapeDtypeStruct((ESPLIT, H), jnp.bfloat16)

    a1, table = _pc(
        _proj_body, GRID_FULL,
        [_aspec(), _wspec(), _bspec(), _wspec()],
        [_aspec(), _aspec()],
        [jax.ShapeDtypeStruct((B, H), jnp.float32)] * 2,
    )(atom, wbe1, b_be2, wbe2)

    idx_s = [lax.slice_in_dim(idx, s * ESPLIT, (s + 1) * ESPLIT)
             for s in range(NSPLIT)]

    bh_s, ah_s, p1_s, p2_s = [], [], [], []
    for s in range(NSPLIT):
        g = _sc_gather(table, idx_s[s])
        bh, ah, p1, p2 = _pc(
            _layer0_body, GRID,
            [pl.BlockSpec((BA, NNN, NBF), lambda i, o=s: (o * GRID + i, 0, 0)),
             _espec(), _aspec(s), _aspec(s), _wspec(NBF),
             _wspec(), _wspec(), _bspec(), _wspec(), _bspec(), _wspec()],
            [_espec(), _aspec(), _aspec(), _aspec()],
            [bh_out, atom_out, atom_out, atom_out],
        )(bonds, g, a1, atom, wbe3, wae1, wae2, b_ae2, wbu1, b_bu2, wbu2)
        bh_s.append(bh); ah_s.append(ah); p1_s.append(p1); p2_s.append(p2)

    for layer in range(3):
        table = jnp.concatenate(p2_s, axis=0)
        last = layer == 2
        new = [[], [], [], []]
        for s in range(NSPLIT):
            g = _sc_gather(table, idx_s[s])
            if last:
                (y,) = _pc(
                    _final_body, GRID,
                    [_espec(), _espec(), _aspec(), _aspec(),
                     _wspec(), _wspec(), _wspec(), _bspec(),
                     pl.BlockSpec((H, 1), lambda i: (0, 0)),
                     pl.BlockSpec((1, 1), lambda i: (0, 0))],
                    [_aspec(width=1)],
                    [jax.ShapeDtypeStruct((ASPLIT, 1), jnp.float32)],
                )(bh_s[s], g, p1_s[s], ah_s[s], wbu3, wau1, wau2, b_au2,
                  W_fc, b_fc2)
                new[0].append(y)
            else:
                bh, ah, p1, p2 = _pc(
                    _conv_body, GRID,
                    [_espec(), _espec(), _aspec(), _aspec(),
                     _wspec(), _wspec(), _wspec(), _bspec(), _wspec(),
                     _bspec(), _wspec()],
                    [_espec(), _aspec(), _aspec(), _aspec()],
                    [bh_out, atom_out, atom_out, atom_out],
                )(bh_s[s], g, p1_s[s], ah_s[s], wbu3, wau1, wau2, b_au2,
                  wbu1, b_bu2, wbu2)
                new[0].append(bh); new[1].append(ah)
                new[2].append(p1); new[3].append(p2)
        if last:
            return jnp.concatenate(new[0], axis=0)
        bh_s, ah_s, p1_s, p2_s = new


# speedup vs baseline: 3.9976x; 1.0313x over previous
import functools

import jax
import jax.numpy as jnp
from jax import lax
from jax.experimental import pallas as pl
from jax.experimental.pallas import tpu as pltpu
from jax.experimental.pallas import tpu_sc as plsc

B = 10000
NNN = 32
E = B * NNN
NBF = 16
H = 128

SPLITS = (4000, 4000, 2000)
NSPLIT = len(SPLITS)
AOFF = [sum(SPLITS[:s]) for s in range(NSPLIT + 1)]

NC = 2
NS = 16
NW = NC * NS
CHUNK = 128
NBUF = 2
LOOKAHEAD = NBUF - 1


def _gather_body(n_edges, table_hbm, idx_hbm, out_hbm, idx_all, tbl, *bufs):
    SPAN = n_edges // NW
    CS = [CHUNK] * (SPAN // CHUNK) + ([SPAN % CHUNK] if SPAN % CHUNK else [])
    OFF = [sum(CS[:k]) for k in range(len(CS))]
    NB = len(CS)
    rows = bufs[:NBUF]
    sg = bufs[NBUF:2 * NBUF]
    so = bufs[2 * NBUF:]
    wid = lax.axis_index("s") * NC + lax.axis_index("c")
    base = wid * SPAN

    @pl.when(lax.axis_index("s") == 0)
    def _():
        pltpu.sync_copy(table_hbm, tbl)

    pltpu.sync_copy(idx_hbm.at[pl.ds(base, SPAN)], idx_all)
    plsc.subcore_barrier()

    def idx_ref(k):
        return idx_all.at[pl.ds(OFF[k], CS[k])]

    def rows_ref(k, q):
        return rows[q] if CS[k] == CHUNK else rows[q].at[pl.ds(0, CS[k])]

    def out_ref(k):
        return out_hbm.at[pl.ds(base + OFF[k], CS[k])]

    def g_start(k, q):
        pltpu.async_copy(tbl.at[idx_ref(k)], rows_ref(k, q), sg[q])

    def g_wait(k, q):
        pltpu.make_async_copy(tbl.at[idx_ref(k)], rows_ref(k, q),
                              sg[q]).wait()

    def o_start(k, q):
        pltpu.async_copy(rows_ref(k, q), out_ref(k), so[q])

    def o_wait(k, q):
        pltpu.make_async_copy(rows_ref(k, q), out_ref(k), so[q]).wait()

    for j in range(min(LOOKAHEAD, NB)):
        g_start(j, j % NBUF)
    for k in range(NB):
        kk = k + LOOKAHEAD
        if kk < NB:
            q = kk % NBUF
            if kk >= NBUF:
                o_wait(kk - NBUF, q)
            g_start(kk, q)
        p = k % NBUF
        g_wait(k, p)
        o_start(k, p)
    for k in range(max(0, NB - NBUF), NB):
        o_wait(k, k % NBUF)


@functools.lru_cache
def _make_sc_gather(n_edges):
    return pl.kernel(
        functools.partial(_gather_body, n_edges),
        out_type=jax.ShapeDtypeStruct((n_edges, H), jnp.float32),
        mesh=plsc.VectorSubcoreMesh(
            core_axis_name="c", subcore_axis_name="s",
            num_cores=NC, num_subcores=NS
        ),
        scratch_types=[pltpu.VMEM((n_edges // NW,), jnp.int32),
                       pltpu.VMEM_SHARED((B, H), jnp.float32)]
        + [pltpu.VMEM((CHUNK, H), jnp.float32)] * NBUF
        + [pltpu.SemaphoreType.DMA] * (2 * NBUF),
    )


def _sc_gather(table, idx_split):
    return _make_sc_gather(idx_split.shape[0])(table, idx_split)

BA = 400
EBLK = BA * NNN
GRID_FULL = B // BA

_dot = functools.partial(jnp.dot, preferred_element_type=jnp.float32)


def _proj_body(atom_ref, w1_ref, b1_ref, w2_ref, a1_ref, a2_ref):
    a = atom_ref[...]
    a1_ref[...] = _dot(a, w1_ref[...]) + b1_ref[...]
    a2_ref[...] = _dot(a, w2_ref[...])


def _edge_atom_stage(m, g, p1, ah, wau1, wau2, bau, t_ref):
    t = jnp.tanh((m + g).reshape(BA, NNN, H) + p1[:, None, :])
    if t_ref is not None:
        t_ref[...] = t.reshape(EBLK, H).astype(t_ref.dtype)
    mean = jnp.sum(t, axis=1) * (1.0 / NNN)
    return jnp.maximum(_dot(mean, wau1) + _dot(ah, wau2) + bau, 0.0)


def _layer0_body(bonds_ref, g_ref, a1_ref, atom_ref, w3_ref, wae1_ref,
                 wae2_ref, bae_ref, wb1_ref, bb_ref, wb2_ref,
                 bh_ref, ah_ref, p1_ref, p2_ref):
    m = _dot(bonds_ref[...].reshape(EBLK, NBF), w3_ref[...])
    ah = _edge_atom_stage(m, g_ref[...], a1_ref[...], atom_ref[...],
                          wae1_ref[...], wae2_ref[...], bae_ref[...], bh_ref)
    ah_ref[...] = ah
    p1_ref[...] = _dot(ah, wb1_ref[...]) + bb_ref[...]
    p2_ref[...] = _dot(ah, wb2_ref[...])


def _conv_body(bhin_ref, g_ref, p1in_ref, ahin_ref, w3_ref, wau1_ref,
               wau2_ref, bau_ref, wb1_ref, bb_ref, wb2_ref,
               bh_ref, ah_ref, p1_ref, p2_ref):
    m = _dot(bhin_ref[...], w3_ref[...])
    ah = _edge_atom_stage(m, g_ref[...], p1in_ref[...], ahin_ref[...],
                          wau1_ref[...], wau2_ref[...], bau_ref[...], bh_ref)
    ah_ref[...] = ah
    p1_ref[...] = _dot(ah, wb1_ref[...]) + bb_ref[...]
    p2_ref[...] = _dot(ah, wb2_ref[...])


def _final_body(bhin_ref, g_ref, p1in_ref, ahin_ref, w3_ref, wau1_ref,
                wau2_ref, bau_ref, wfc_ref, bfc_ref, y_ref):
    m = _dot(bhin_ref[...], w3_ref[...])
    ah = _edge_atom_stage(m, g_ref[...], p1in_ref[...], ahin_ref[...],
                          wau1_ref[...], wau2_ref[...], bau_ref[...], None)
    z = _dot(ah, wfc_ref[...]) + bfc_ref[...]
    y_ref[...] = jnp.maximum(z, 0.0) + jnp.log1p(jnp.exp(-jnp.abs(z)))


def _espec(blk_off=0):
    return pl.BlockSpec((EBLK, H), lambda i, o=blk_off: (o + i, 0))


def _aspec(blk_off=0, width=H):
    return pl.BlockSpec((BA, width), lambda i, o=blk_off: (o + i, 0))


def _wspec(rows=H):
    return pl.BlockSpec((rows, H), lambda i: (0, 0))


def _bspec():
    return pl.BlockSpec((1, H), lambda i: (0, 0))


_params = pltpu.CompilerParams(dimension_semantics=("parallel",))


def _pc(body, grid, in_specs, out_specs, out_shapes):
    return pl.pallas_call(
        body,
        grid=(grid,),
        in_specs=in_specs,
        out_specs=out_specs,
        out_shape=out_shapes,
        compiler_params=_params,
    )


def kernel(gmap, atom, bonds, W_be, b_be, W_ae, b_ae, W_bu, b_bu, W_au, b_au,
           W_fc, b_fc):
    idx = gmap.astype(jnp.int32).reshape(E)

    wbe1, wbe2, wbe3 = W_be[:H], W_be[H:2 * H], W_be[2 * H:]
    wae1, wae2 = W_ae[:H], W_ae[H:]
    wbu1, wbu2 = W_bu[:H], W_bu[H:2 * H]
    wbu3 = W_bu[2 * H:].astype(jnp.bfloat16)
    wau1, wau2 = W_au[:H], W_au[H:]
    b_be2 = b_be.reshape(1, H)
    b_ae2 = b_ae.reshape(1, H)
    b_bu2 = b_bu.reshape(1, H)
    b_au2 = b_au.reshape(1, H)
    b_fc2 = b_fc.reshape(1, 1)

    a1, table = _pc(
        _proj_body, GRID_FULL,
        [_aspec(), _wspec(), _bspec(), _wspec()],
        [_aspec(), _aspec()],
        [jax.ShapeDtypeStruct((B, H), jnp.float32)] * 2,
    )(atom, wbe1, b_be2, wbe2)

    idx_s = [lax.slice_in_dim(idx, AOFF[s] * NNN, AOFF[s + 1] * NNN)
             for s in range(NSPLIT)]

    def split_shapes(s):
        na = SPLITS[s]
        return (na // BA,
                jax.ShapeDtypeStruct((na, H), jnp.float32),
                jax.ShapeDtypeStruct((na * NNN, H), jnp.bfloat16))

    bh_s, ah_s, p1_s, p2_s = [], [], [], []
    for s in range(NSPLIT):
        grid, atom_out, bh_out = split_shapes(s)
        blk = AOFF[s] // BA
        g = _sc_gather(table, idx_s[s])
        bh, ah, p1, p2 = _pc(
            _layer0_body, grid,
            [pl.BlockSpec((BA, NNN, NBF), lambda i, o=blk: (o + i, 0, 0)),
             _espec(), _aspec(blk), _aspec(blk), _wspec(NBF),
             _wspec(), _wspec(), _bspec(), _wspec(), _bspec(), _wspec()],
            [_espec(), _aspec(), _aspec(), _aspec()],
            [bh_out, atom_out, atom_out, atom_out],
        )(bonds, g, a1, atom, wbe3, wae1, wae2, b_ae2, wbu1, b_bu2, wbu2)
        bh_s.append(bh); ah_s.append(ah); p1_s.append(p1); p2_s.append(p2)

    for layer in range(3):
        table = jnp.concatenate(p2_s, axis=0)
        last = layer == 2
        new = [[], [], [], []]
        for s in range(NSPLIT):
            grid, atom_out, bh_out = split_shapes(s)
            g = _sc_gather(table, idx_s[s])
            if last:
                (y,) = _pc(
                    _final_body, grid,
                    [_espec(), _espec(), _aspec(), _aspec(),
                     _wspec(), _wspec(), _wspec(), _bspec(),
                     pl.BlockSpec((H, 1), lambda i: (0, 0)),
                     pl.BlockSpec((1, 1), lambda i: (0, 0))],
                    [_aspec(width=1)],
                    [jax.ShapeDtypeStruct((SPLITS[s], 1), jnp.float32)],
                )(bh_s[s], g, p1_s[s], ah_s[s], wbu3, wau1, wau2, b_au2,
                  W_fc, b_fc2)
                new[0].append(y)
            else:
                bh, ah, p1, p2 = _pc(
                    _conv_body, grid,
                    [_espec(), _espec(), _aspec(), _aspec(),
                     _wspec(), _wspec(), _wspec(), _bspec(), _wspec(),
                     _bspec(), _wspec()],
                    [_espec(), _aspec(), _aspec(), _aspec()],
                    [bh_out, atom_out, atom_out, atom_out],
                )(bh_s[s], g, p1_s[s], ah_s[s], wbu3, wau1, wau2, b_au2,
                  wbu1, b_bu2, wbu2)
                new[0].append(bh); new[1].append(ah)
                new[2].append(p1); new[3].append(p2)
        if last:
            return jnp.concatenate(new[0], axis=0)
        bh_s, ah_s, p1_s, p2_s = new


# speedup vs baseline: 4.0846x; 1.0218x over previous
import functools

import jax
import jax.numpy as jnp
from jax import lax
from jax.experimental import pallas as pl
from jax.experimental.pallas import tpu as pltpu
from jax.experimental.pallas import tpu_sc as plsc

B = 10000
NNN = 32
E = B * NNN
NBF = 16
H = 128

SPLITS = (4000, 4400, 1600)
NSPLIT = len(SPLITS)
AOFF = [sum(SPLITS[:s]) for s in range(NSPLIT + 1)]

NC = 2
NS = 16
NW = NC * NS
CHUNK = 128
NBUF = 2
LOOKAHEAD = NBUF - 1


def _gather_body(n_edges, tbl_sizes, *refs):
    ntbl = len(tbl_sizes)
    tbl_hbm = refs[:ntbl]
    idx_hbm, out_hbm, idx_all, tbl = refs[ntbl:ntbl + 4]
    bufs = refs[ntbl + 4:]
    SPAN = n_edges // NW
    CS = [CHUNK] * (SPAN // CHUNK) + ([SPAN % CHUNK] if SPAN % CHUNK else [])
    OFF = [sum(CS[:k]) for k in range(len(CS))]
    NB = len(CS)
    rows = bufs[:NBUF]
    sg = bufs[NBUF:2 * NBUF]
    so = bufs[2 * NBUF:]
    sid = lax.axis_index("s")
    wid = sid * NC + lax.axis_index("c")
    base = wid * SPAN

    toff = 0
    for t in range(ntbl):
        @pl.when(sid == t)
        def _(t=t, toff=toff):
            pltpu.sync_copy(tbl_hbm[t], tbl.at[pl.ds(toff, tbl_sizes[t])])
        toff += tbl_sizes[t]

    pltpu.sync_copy(idx_hbm.at[pl.ds(base, SPAN)], idx_all)
    plsc.subcore_barrier()

    def idx_ref(k):
        return idx_all.at[pl.ds(OFF[k], CS[k])]

    def rows_ref(k, q):
        return rows[q] if CS[k] == CHUNK else rows[q].at[pl.ds(0, CS[k])]

    def out_ref(k):
        return out_hbm.at[pl.ds(base + OFF[k], CS[k])]

    def g_start(k, q):
        pltpu.async_copy(tbl.at[idx_ref(k)], rows_ref(k, q), sg[q])

    def g_wait(k, q):
        pltpu.make_async_copy(tbl.at[idx_ref(k)], rows_ref(k, q),
                              sg[q]).wait()

    def o_start(k, q):
        pltpu.async_copy(rows_ref(k, q), out_ref(k), so[q])

    def o_wait(k, q):
        pltpu.make_async_copy(rows_ref(k, q), out_ref(k), so[q]).wait()

    for j in range(min(LOOKAHEAD, NB)):
        g_start(j, j % NBUF)
    for k in range(NB):
        kk = k + LOOKAHEAD
        if kk < NB:
            q = kk % NBUF
            if kk >= NBUF:
                o_wait(kk - NBUF, q)
            g_start(kk, q)
        p = k % NBUF
        g_wait(k, p)
        o_start(k, p)
    for k in range(max(0, NB - NBUF), NB):
        o_wait(k, k % NBUF)


@functools.lru_cache
def _make_sc_gather(n_edges, tbl_sizes):
    return pl.kernel(
        functools.partial(_gather_body, n_edges, tbl_sizes),
        out_type=jax.ShapeDtypeStruct((n_edges, H), jnp.float32),
        mesh=plsc.VectorSubcoreMesh(
            core_axis_name="c", subcore_axis_name="s",
            num_cores=NC, num_subcores=NS
        ),
        scratch_types=[pltpu.VMEM((n_edges // NW,), jnp.int32),
                       pltpu.VMEM_SHARED((B, H), jnp.float32)]
        + [pltpu.VMEM((CHUNK, H), jnp.float32)] * NBUF
        + [pltpu.SemaphoreType.DMA] * (2 * NBUF),
    )


def _sc_gather(tables, idx_split):
    sizes = tuple(t.shape[0] for t in tables)
    return _make_sc_gather(idx_split.shape[0], sizes)(*tables, idx_split)

BA = 400
EBLK = BA * NNN
GRID_FULL = B // BA

_dot = functools.partial(jnp.dot, preferred_element_type=jnp.float32)


def _proj_body(atom_ref, w1_ref, b1_ref, w2_ref, a1_ref, a2_ref):
    a = atom_ref[...]
    a1_ref[...] = _dot(a, w1_ref[...]) + b1_ref[...]
    a2_ref[...] = _dot(a, w2_ref[...])


def _edge_atom_stage(m, g, p1, ah, wau1, wau2, bau, t_ref):
    t = jnp.tanh((m + g).reshape(BA, NNN, H) + p1[:, None, :])
    if t_ref is not None:
        t_ref[...] = t.reshape(EBLK, H).astype(t_ref.dtype)
    mean = jnp.sum(t, axis=1) * (1.0 / NNN)
    return jnp.maximum(_dot(mean, wau1) + _dot(ah, wau2) + bau, 0.0)


def _layer0_body(bonds_ref, g_ref, a1_ref, atom_ref, w3_ref, wae1_ref,
                 wae2_ref, bae_ref, wb1_ref, bb_ref, wb2_ref,
                 bh_ref, ah_ref, p1_ref, p2_ref):
    m = _dot(bonds_ref[...].reshape(EBLK, NBF), w3_ref[...])
    ah = _edge_atom_stage(m, g_ref[...], a1_ref[...], atom_ref[...],
                          wae1_ref[...], wae2_ref[...], bae_ref[...], bh_ref)
    ah_ref[...] = ah
    p1_ref[...] = _dot(ah, wb1_ref[...]) + bb_ref[...]
    p2_ref[...] = _dot(ah, wb2_ref[...])


def _conv_body(bhin_ref, g_ref, p1in_ref, ahin_ref, w3_ref, wau1_ref,
               wau2_ref, bau_ref, wb1_ref, bb_ref, wb2_ref,
               bh_ref, ah_ref, p1_ref, p2_ref):
    m = _dot(bhin_ref[...], w3_ref[...])
    ah = _edge_atom_stage(m, g_ref[...], p1in_ref[...], ahin_ref[...],
                          wau1_ref[...], wau2_ref[...], bau_ref[...], bh_ref)
    ah_ref[...] = ah
    p1_ref[...] = _dot(ah, wb1_ref[...]) + bb_ref[...]
    p2_ref[...] = _dot(ah, wb2_ref[...])


def _final_body(bhin_ref, g_ref, p1in_ref, ahin_ref, w3_ref, wau1_ref,
                wau2_ref, bau_ref, wfc_ref, bfc_ref, y_ref):
    m = _dot(bhin_ref[...], w3_ref[...])
    ah = _edge_atom_stage(m, g_ref[...], p1in_ref[...], ahin_ref[...],
                          wau1_ref[...], wau2_ref[...], bau_ref[...], None)
    z = _dot(ah, wfc_ref[...]) + bfc_ref[...]
    y_ref[...] = jnp.maximum(z, 0.0) + jnp.log1p(jnp.exp(-jnp.abs(z)))


def _espec(blk_off=0):
    return pl.BlockSpec((EBLK, H), lambda i, o=blk_off: (o + i, 0))


def _aspec(blk_off=0, width=H):
    return pl.BlockSpec((BA, width), lambda i, o=blk_off: (o + i, 0))


def _wspec(rows=H):
    return pl.BlockSpec((rows, H), lambda i: (0, 0))


def _bspec():
    return pl.BlockSpec((1, H), lambda i: (0, 0))


_params = pltpu.CompilerParams(dimension_semantics=("parallel",))


def _pc(body, grid, in_specs, out_specs, out_shapes):
    return pl.pallas_call(
        body,
        grid=(grid,),
        in_specs=in_specs,
        out_specs=out_specs,
        out_shape=out_shapes,
        compiler_params=_params,
    )


def kernel(gmap, atom, bonds, W_be, b_be, W_ae, b_ae, W_bu, b_bu, W_au, b_au,
           W_fc, b_fc):
    idx = gmap.astype(jnp.int32).reshape(E)

    wbe1, wbe2, wbe3 = W_be[:H], W_be[H:2 * H], W_be[2 * H:]
    wae1, wae2 = W_ae[:H], W_ae[H:]
    wbu1, wbu2 = W_bu[:H], W_bu[H:2 * H]
    wbu3 = W_bu[2 * H:].astype(jnp.bfloat16)
    wau1, wau2 = W_au[:H], W_au[H:]
    b_be2 = b_be.reshape(1, H)
    b_ae2 = b_ae.reshape(1, H)
    b_bu2 = b_bu.reshape(1, H)
    b_au2 = b_au.reshape(1, H)
    b_fc2 = b_fc.reshape(1, 1)

    a1, table = _pc(
        _proj_body, GRID_FULL,
        [_aspec(), _wspec(), _bspec(), _wspec()],
        [_aspec(), _aspec()],
        [jax.ShapeDtypeStruct((B, H), jnp.float32)] * 2,
    )(atom, wbe1, b_be2, wbe2)

    idx_s = [lax.slice_in_dim(idx, AOFF[s] * NNN, AOFF[s + 1] * NNN)
             for s in range(NSPLIT)]

    def split_shapes(s):
        na = SPLITS[s]
        return (na // BA,
                jax.ShapeDtypeStruct((na, H), jnp.float32),
                jax.ShapeDtypeStruct((na * NNN, H), jnp.bfloat16))

    bh_s, ah_s, p1_s, p2_s = [], [], [], []
    for s in range(NSPLIT):
        grid, atom_out, bh_out = split_shapes(s)
        blk = AOFF[s] // BA
        g = _sc_gather((table,), idx_s[s])
        bh, ah, p1, p2 = _pc(
            _layer0_body, grid,
            [pl.BlockSpec((BA, NNN, NBF), lambda i, o=blk: (o + i, 0, 0)),
             _espec(), _aspec(blk), _aspec(blk), _wspec(NBF),
             _wspec(), _wspec(), _bspec(), _wspec(), _bspec(), _wspec()],
            [_espec(), _aspec(), _aspec(), _aspec()],
            [bh_out, atom_out, atom_out, atom_out],
        )(bonds, g, a1, atom, wbe3, wae1, wae2, b_ae2, wbu1, b_bu2, wbu2)
        bh_s.append(bh); ah_s.append(ah); p1_s.append(p1); p2_s.append(p2)

    for layer in range(3):
        last = layer == 2
        new = [[], [], [], []]
        for s in range(NSPLIT):
            grid, atom_out, bh_out = split_shapes(s)
            g = _sc_gather(tuple(p2_s), idx_s[s])
            if last:
                (y,) = _pc(
                    _final_body, grid,
                    [_espec(), _espec(), _aspec(), _aspec(),
                     _wspec(), _wspec(), _wspec(), _bspec(),
                     pl.BlockSpec((H, 1), lambda i: (0, 0)),
                     pl.BlockSpec((1, 1), lambda i: (0, 0))],
                    [_aspec(width=1)],
                    [jax.ShapeDtypeStruct((SPLITS[s], 1), jnp.float32)],
                )(bh_s[s], g, p1_s[s], ah_s[s], wbu3, wau1, wau2, b_au2,
                  W_fc, b_fc2)
                new[0].append(y)
            else:
                bh, ah, p1, p2 = _pc(
                    _conv_body, grid,
                    [_espec(), _espec(), _aspec(), _aspec(),
                     _wspec(), _wspec(), _wspec(), _bspec(), _wspec(),
                     _bspec(), _wspec()],
                    [_espec(), _aspec(), _aspec(), _aspec()],
                    [bh_out, atom_out, atom_out, atom_out],
                )(bh_s[s], g, p1_s[s], ah_s[s], wbu3, wau1, wau2, b_au2,
                  wbu1, b_bu2, wbu2)
                new[0].append(bh); new[1].append(ah)
                new[2].append(p1); new[3].append(p2)
        if last:
            return jnp.concatenate(new[0], axis=0)
        bh_s, ah_s, p1_s, p2_s = new
